# Initial kernel scaffold; baseline (speedup 1.0000x reference)
import jax
import jax.numpy as jnp
from jax.experimental import pallas as pl


def kernel(x, W_ih, W_hh, b_ih, b_hh, Wxh, bxh, Whh_r, W1, b1, W2, b2):
    raise NotImplementedError("write your pallas kernel here")


# speedup vs baseline: 2.3051x; 2.3051x over previous
import jax
import jax.numpy as jnp
from jax.experimental import pallas as pl
from jax.experimental.pallas import tpu as pltpu

B, T, D_IN = 256, 32, 512
H = 512
OUT = 512
E = 16
K = 2
FF = 2048
W_IMP = 0.01
W_LOAD = 0.01


def _lstm_body(xt_ref, wih_ref, whh_ref, b_ref, wxh_ref, bxh_ref, wr_ref,
               hT_ref, logits_ref, h_scr, c_scr, r_scr):
    t = pl.program_id(0)

    @pl.when(t == 0)
    def _init():
        h_scr[...] = jnp.zeros_like(h_scr)
        c_scr[...] = jnp.zeros_like(c_scr)
        r_scr[...] = jnp.zeros_like(r_scr)

    xt = xt_ref[0]
    h = h_scr[...]
    g = (jnp.dot(xt, wih_ref[...], preferred_element_type=jnp.float32)
         + jnp.dot(h, whh_ref[...], preferred_element_type=jnp.float32)
         + b_ref[...])
    i = jax.nn.sigmoid(g[:, 0:H])
    f = jax.nn.sigmoid(g[:, H:2 * H])
    gg = jnp.tanh(g[:, 2 * H:3 * H])
    o = jax.nn.sigmoid(g[:, 3 * H:4 * H])
    c = f * c_scr[...] + i * gg
    h = o * jnp.tanh(c)
    h_scr[...] = h
    c_scr[...] = c
    r = (jnp.dot(h, wxh_ref[...], preferred_element_type=jnp.float32)
         + bxh_ref[...]
         + jnp.dot(r_scr[...], wr_ref[...], preferred_element_type=jnp.float32))
    r_scr[...] = r

    @pl.when(t == T - 1)
    def _emit():
        hT_ref[...] = h
        logits_ref[...] = r


def _gate_body(logits_ref, gates_ref, loss_ref):
    l = logits_ref[...]
    col = jax.lax.broadcasted_iota(jnp.int32, (B, E), 1)
    m1 = jnp.max(l, axis=1, keepdims=True)
    idx1 = jnp.min(jnp.where(l == m1, col, E), axis=1, keepdims=True)
    pick1 = col == idx1
    l2 = jnp.where(pick1, -jnp.inf, l)
    m2 = jnp.max(l2, axis=1, keepdims=True)
    idx2 = jnp.min(jnp.where(l2 == m2, col, E), axis=1, keepdims=True)
    pick2 = col == idx2
    e2 = jnp.exp(m2 - m1)
    denom = 1.0 + e2
    p1 = 1.0 / denom
    p2 = e2 / denom
    gates = jnp.where(pick1, p1, 0.0) + jnp.where(pick2, p2, 0.0)
    gates_ref[...] = gates

    importance = jnp.sum(gates, axis=0)
    load = jnp.sum((gates > 0.0).astype(jnp.float32), axis=0)

    def cv2(v):
        m = jnp.mean(v)
        var = jnp.sum((v - m) ** 2) / (E - 1)
        return var / (m * m + 1e-10)

    loss = W_IMP * cv2(importance) + W_LOAD * cv2(load)
    loss_ref[...] = jnp.full((1, 1), loss, jnp.float32)


def _expert_body(hT_ref, gates_ref, g3_ref, w1_ref, b1_ref, w2_ref, b2_ref,
                 y_ref, acc_scr):
    e = pl.program_id(0)

    @pl.when(e == 0)
    def _init():
        acc_scr[...] = jnp.dot(gates_ref[...], b2_ref[...],
                               preferred_element_type=jnp.float32)

    ge = g3_ref[0]
    hidden = jnp.maximum(
        jnp.dot(hT_ref[...], w1_ref[0], preferred_element_type=jnp.float32)
        + b1_ref[0], 0.0)
    acc_scr[...] += jnp.dot(ge * hidden, w2_ref[0],
                            preferred_element_type=jnp.float32)

    @pl.when(e == E - 1)
    def _emit():
        y_ref[...] = acc_scr[...]


def kernel(x, W_ih, W_hh, b_ih, b_hh, Wxh, bxh, Whh_r, W1, b1, W2, b2):
    xt = jnp.swapaxes(x, 0, 1)
    wih = W_ih.T
    whh = W_hh.T
    b = (b_ih + b_hh).reshape(1, 4 * H)
    wxh = Wxh.T
    bxh2 = bxh.reshape(1, E)
    wr = Whh_r.T

    hT, logits = pl.pallas_call(
        _lstm_body,
        grid=(T,),
        in_specs=[
            pl.BlockSpec((1, B, D_IN), lambda t: (t, 0, 0)),
            pl.BlockSpec((D_IN, 4 * H), lambda t: (0, 0)),
            pl.BlockSpec((H, 4 * H), lambda t: (0, 0)),
            pl.BlockSpec((1, 4 * H), lambda t: (0, 0)),
            pl.BlockSpec((H, E), lambda t: (0, 0)),
            pl.BlockSpec((1, E), lambda t: (0, 0)),
            pl.BlockSpec((E, E), lambda t: (0, 0)),
        ],
        out_specs=[
            pl.BlockSpec((B, H), lambda t: (0, 0)),
            pl.BlockSpec((B, E), lambda t: (0, 0)),
        ],
        out_shape=[
            jax.ShapeDtypeStruct((B, H), jnp.float32),
            jax.ShapeDtypeStruct((B, E), jnp.float32),
        ],
        scratch_shapes=[
            pltpu.VMEM((B, H), jnp.float32),
            pltpu.VMEM((B, H), jnp.float32),
            pltpu.VMEM((B, E), jnp.float32),
        ],
    )(xt, wih, whh, b, wxh, bxh2, wr)

    gates, loss2d = pl.pallas_call(
        _gate_body,
        out_shape=[
            jax.ShapeDtypeStruct((B, E), jnp.float32),
            jax.ShapeDtypeStruct((1, 1), jnp.float32),
        ],
    )(logits)

    g3 = gates.T.reshape(E, B, 1)

    y_pred = pl.pallas_call(
        _expert_body,
        grid=(E,),
        in_specs=[
            pl.BlockSpec((B, H), lambda e: (0, 0)),
            pl.BlockSpec((B, E), lambda e: (0, 0)),
            pl.BlockSpec((1, B, 1), lambda e: (e, 0, 0)),
            pl.BlockSpec((1, H, FF), lambda e: (e, 0, 0)),
            pl.BlockSpec((1, 1, FF), lambda e: (e, 0, 0)),
            pl.BlockSpec((1, FF, OUT), lambda e: (e, 0, 0)),
            pl.BlockSpec((E, OUT), lambda e: (0, 0)),
        ],
        out_specs=pl.BlockSpec((B, OUT), lambda e: (0, 0)),
        out_shape=jax.ShapeDtypeStruct((B, OUT), jnp.float32),
        scratch_shapes=[pltpu.VMEM((B, OUT), jnp.float32)],
    )(hT, gates, g3, W1, b1.reshape(E, 1, FF), W2, b2)

    return y_pred, loss2d.reshape(())
